# baseline (speedup 1.0000x reference)
import functools

import jax
import jax.numpy as jnp
from jax import lax
from jax.experimental import pallas as pl
from jax.experimental.pallas import tpu as pltpu
from jax.experimental.pallas import tpu_sc as plsc

L = 16
NC = 2
NS = 16
NW = NC * NS
MAX_IDX = 128


def _chunks(h):
    out = []
    off = 0
    while off < h:
        n = min(MAX_IDX, h - off)
        out.append((off, n))
        off += n
    return out


@functools.lru_cache(maxsize=None)
def _build(B, H, V, D):
    assert B % NW == 0, (B, NW)
    bpw = B // NW
    assert bpw % 2 == 0
    nd = D // L
    chunks = _chunks(H)
    mesh = plsc.VectorSubcoreMesh(core_axis_name="c", subcore_axis_name="s")

    def body(x_hbm, tab_hbm, out_hbm, idx_v, rows0, rows1, out_v, sem0, sem1):
        wid = lax.axis_index("s") * NC + lax.axis_index("c")
        base = wid * bpw
        pltpu.sync_copy(x_hbm.at[pl.ds(base, bpw)], idx_v)

        def start(i, rows, sem):
            for off, n in chunks:
                pltpu.async_copy(
                    tab_hbm.at[idx_v.at[i, pl.ds(off, n)]],
                    rows.at[pl.ds(off, n)],
                    sem,
                )

        def drain(i, rows, sem):
            for off, n in chunks:
                pltpu.make_async_copy(
                    tab_hbm.at[idx_v.at[i, pl.ds(off, n)]],
                    rows.at[pl.ds(off, n)],
                    sem,
                ).wait()

        ones = jnp.ones((L,), jnp.float32)
        neg_ones = -ones

        def consume(i, rows):
            def rbody(r, acc):
                return tuple(
                    acc[c] + rows[r, pl.ds(c * L, L)] for c in range(nd)
                )

            acc = lax.fori_loop(
                0, H, rbody,
                tuple(jnp.zeros((L,), jnp.float32) for _ in range(nd)),
                unroll=4,
            )
            for c in range(nd):
                out_v[i, pl.ds(c * L, L)] = lax.select(
                    acc[c] > 0.0, ones, neg_ones
                )

        start(0, rows0, sem0)
        start(1, rows1, sem1)

        def gbody(g, _):
            i0 = 2 * g
            drain(i0, rows0, sem0)
            consume(i0, rows0)

            @pl.when(g < bpw // 2 - 1)
            def _():
                start(i0 + 2, rows0, sem0)

            drain(i0 + 1, rows1, sem1)
            consume(i0 + 1, rows1)

            @pl.when(g < bpw // 2 - 1)
            def _():
                start(i0 + 3, rows1, sem1)

            return 0

        lax.fori_loop(0, bpw // 2, gbody, 0)
        pltpu.sync_copy(out_v, out_hbm.at[pl.ds(base, bpw)])

    return pl.kernel(
        body,
        out_type=jax.ShapeDtypeStruct((B, D), jnp.float32),
        mesh=mesh,
        compiler_params=pltpu.CompilerParams(use_tc_tiling_on_sc=False),
        scratch_types=[
            pltpu.VMEM((bpw, H), jnp.int32),
            pltpu.VMEM((H, D), jnp.float32),
            pltpu.VMEM((H, D), jnp.float32),
            pltpu.VMEM((bpw, D), jnp.float32),
            pltpu.SemaphoreType.DMA,
            pltpu.SemaphoreType.DMA,
        ],
    )


def kernel(x, embed_weight):
    B, H = x.shape
    V, D = embed_weight.shape
    return _build(B, H, V, D)(x, embed_weight)
